# baseline (speedup 1.0000x reference)
import functools

import jax
import jax.numpy as jnp
from jax.experimental import pallas as pl
from jax.experimental.pallas import tpu as pltpu

_BQ = 128
_K = 10
_BIG_I = 2**30
_BIG_F = 1e30


def _knn_kernel(q_ref, s_ref, sd_ref, idx_ref, *, ns):
    q = q_ref[...]
    s = s_ref[...]
    sx = s[0:3, :]
    sn = s[3:6, :]
    r_s = jnp.sum(sx * sx, axis=0, keepdims=True)
    c_s = jnp.sum(sn * sx, axis=0, keepdims=True)

    lane_mask = (jax.lax.broadcasted_iota(jnp.int32, (1, 8), 1) < 3)
    q_xyz8 = jnp.where(lane_mask, q, 0.0)
    r_q = jnp.sum(q_xyz8 * q_xyz8, axis=1, keepdims=True)

    mul = jax.lax.dot_general(
        q_xyz8, s, (((1,), (0,)), ((), ())),
        preferred_element_type=jnp.float32,
        precision=jax.lax.Precision.DEFAULT)
    d2 = r_q - 2.0 * mul + r_s

    nq_mask = (jax.lax.broadcasted_iota(jnp.int32, (1, 8), 1) >= 3) & (
        jax.lax.broadcasted_iota(jnp.int32, (1, 8), 1) < 6)
    q_roll = jnp.roll(q_xyz8, 3, axis=1)
    q_neg = jnp.where(nq_mask, -q_roll, 0.0)
    iv = jax.lax.dot_general(
        q_neg, s, (((1,), (0,)), ((), ())),
        preferred_element_type=jnp.float32,
        precision=jax.lax.Precision.HIGHEST) + c_s

    iota = jax.lax.broadcasted_iota(jnp.int32, (_BQ, ns), 1)
    code = iota * 2 + (iv > 0.0).astype(jnp.int32)

    count = jnp.zeros((_BQ, 1), jnp.int32)
    idx_cols = []
    d0 = None
    for t in range(_K):
        m = jnp.min(d2, axis=1, keepdims=True)
        mc = jnp.where(d2 == m, code, _BIG_I)
        ct = jnp.min(mc, axis=1, keepdims=True)
        idx_cols.append(jax.lax.shift_right_logical(ct, 1))
        count = count + (ct & 1)
        if t == 0:
            d0 = m
        d2 = jnp.where(mc == ct, _BIG_F, d2)

    dist = jnp.sqrt(jnp.maximum(d0, 1e-12))
    inside = count > 8
    distance = jnp.where(inside, -dist, dist)
    qz = q[:, 2:3]
    sd_ref[...] = jnp.minimum(qz, distance)
    idx_ref[...] = jnp.concatenate(idx_cols, axis=1)


@jax.jit
def _run(points_a, points_b):
    ns = points_a.shape[0]
    nq = points_b.shape[0]
    s = jnp.zeros((8, ns), jnp.float32).at[0:6, :].set(points_a.T)
    q = jnp.zeros((nq, 8), jnp.float32).at[:, 0:6].set(points_b)
    grid = nq // _BQ
    sd, idx = pl.pallas_call(
        functools.partial(_knn_kernel, ns=ns),
        grid=(grid,),
        in_specs=[
            pl.BlockSpec((_BQ, 8), lambda i: (i, 0)),
            pl.BlockSpec((8, ns), lambda i: (0, 0)),
        ],
        out_specs=[
            pl.BlockSpec((_BQ, 1), lambda i: (i, 0)),
            pl.BlockSpec((_BQ, _K), lambda i: (i, 0)),
        ],
        out_shape=[
            jax.ShapeDtypeStruct((nq, 1), jnp.float32),
            jax.ShapeDtypeStruct((nq, _K), jnp.int32),
        ],
        compiler_params=pltpu.CompilerParams(
            dimension_semantics=("parallel",)),
    )(q, s)
    return sd[:, 0], idx


def kernel(points_a, points_b, k):
    del k
    return _run(points_a, points_b)


# speedup vs baseline: 1.1835x; 1.1835x over previous
import functools

import jax
import jax.numpy as jnp
from jax import lax
from jax.experimental import pallas as pl
from jax.experimental.pallas import tpu as pltpu
from jax.experimental.pallas import tpu_sc as plsc

_BQ = 128
_K = 10
_BIG_I = 2**30
_BIG_F = 1e30


def _knn_kernel(q_ref, s_ref, dist_ref, idx_ref, c_ref, *, ns):
    q = q_ref[...]
    s = s_ref[...]
    sx = s[0:3, :]
    sn = s[3:6, :]
    r_s = jnp.sum(sx * sx, axis=0, keepdims=True)
    c_ref[...] = jnp.sum(sn * sx, axis=0, keepdims=True)

    lane_mask = (lax.broadcasted_iota(jnp.int32, (1, 8), 1) < 3)
    q_xyz8 = jnp.where(lane_mask, q, 0.0)
    r_q = jnp.sum(q_xyz8 * q_xyz8, axis=1, keepdims=True)

    mul = lax.dot_general(
        q_xyz8, s, (((1,), (0,)), ((), ())),
        preferred_element_type=jnp.float32,
        precision=lax.Precision.DEFAULT)
    d2 = r_q - 2.0 * mul + r_s

    code = lax.broadcasted_iota(jnp.int32, (_BQ, ns), 1)

    idx_cols = []
    d0 = None
    for t in range(_K):
        m = jnp.min(d2, axis=1, keepdims=True)
        mc = jnp.where(d2 == m, code, _BIG_I)
        ct = jnp.min(mc, axis=1, keepdims=True)
        idx_cols.append(ct)
        if t == 0:
            d0 = m
        d2 = jnp.where(mc == ct, _BIG_F, d2)

    dist_ref[...] = jnp.sqrt(jnp.maximum(d0, 1e-12))
    idx_ref[...] = jnp.concatenate(idx_cols, axis=1)


def _sc_sign_kernel(nx_hbm, ny_hbm, nz_hbm, c_hbm, idxf_hbm, qx_hbm,
                    qy_hbm, qz_hbm, dist_hbm, out_hbm,
                    nx_v, ny_v, nz_v, c_v, idx_v, qx_v, qy_v, qz_v,
                    dist_v, out_v, *, nc, nw, nq):
    qpw = nq // nw
    wid = lax.axis_index("s") * nc + lax.axis_index("c")
    base = wid * qpw

    pltpu.sync_copy(nx_hbm, nx_v)
    pltpu.sync_copy(ny_hbm, ny_v)
    pltpu.sync_copy(nz_hbm, nz_v)
    pltpu.sync_copy(c_hbm, c_v)
    for j in range(_K):
        pltpu.sync_copy(idxf_hbm.at[pl.ds(j * nq + base, qpw)],
                        idx_v.at[pl.ds(j * qpw, qpw)])
    pltpu.sync_copy(qx_hbm.at[pl.ds(base, qpw)], qx_v)
    pltpu.sync_copy(qy_hbm.at[pl.ds(base, qpw)], qy_v)
    pltpu.sync_copy(qz_hbm.at[pl.ds(base, qpw)], qz_v)
    pltpu.sync_copy(dist_hbm.at[pl.ds(base, qpw)], dist_v)

    for i in range(qpw // 16):
        sl = pl.ds(i * 16, 16)
        qx = qx_v[sl]
        qy = qy_v[sl]
        qz = qz_v[sl]
        count = jnp.zeros((16,), jnp.int32)
        for j in range(_K):
            iv = idx_v[pl.ds(j * qpw + i * 16, 16)]
            gnx = plsc.load_gather(nx_v, [iv])
            gny = plsc.load_gather(ny_v, [iv])
            gnz = plsc.load_gather(nz_v, [iv])
            gc = plsc.load_gather(c_v, [iv])
            val = gc - (gnx * qx + gny * qy + gnz * qz)
            count = count + jnp.where(val > 0.0, 1, 0)
        dist = dist_v[sl]
        signed = jnp.where(count > 8, -dist, dist)
        out_v[sl] = jnp.minimum(qz, signed)
    pltpu.sync_copy(out_v, out_hbm.at[pl.ds(base, qpw)])


@jax.jit
def _run(points_a, points_b):
    ns = points_a.shape[0]
    nq = points_b.shape[0]
    s = jnp.zeros((8, ns), jnp.float32).at[0:6, :].set(points_a.T)
    q = jnp.zeros((nq, 8), jnp.float32).at[:, 0:6].set(points_b)
    grid = nq // _BQ
    dist, idx, c_row = pl.pallas_call(
        functools.partial(_knn_kernel, ns=ns),
        grid=(grid,),
        in_specs=[
            pl.BlockSpec((_BQ, 8), lambda i: (i, 0)),
            pl.BlockSpec((8, ns), lambda i: (0, 0)),
        ],
        out_specs=[
            pl.BlockSpec((_BQ, 1), lambda i: (i, 0)),
            pl.BlockSpec((_BQ, _K), lambda i: (i, 0)),
            pl.BlockSpec((1, ns), lambda i: (0, 0)),
        ],
        out_shape=[
            jax.ShapeDtypeStruct((nq, 1), jnp.float32),
            jax.ShapeDtypeStruct((nq, _K), jnp.int32),
            jax.ShapeDtypeStruct((1, ns), jnp.float32),
        ],
        compiler_params=pltpu.CompilerParams(
            dimension_semantics=("parallel",)),
    )(q, s)

    info = plsc.get_sparse_core_info()
    nw = info.num_cores * info.num_subcores
    qpw = nq // nw
    idx_f = idx.T.reshape(-1)
    mesh = plsc.VectorSubcoreMesh(core_axis_name="c", subcore_axis_name="s")
    sc = pl.kernel(
        functools.partial(_sc_sign_kernel, nc=info.num_cores, nw=nw, nq=nq),
        mesh=mesh,
        out_type=jax.ShapeDtypeStruct((nq,), jnp.float32),
        scratch_types=[
            pltpu.VMEM((ns,), jnp.float32),
            pltpu.VMEM((ns,), jnp.float32),
            pltpu.VMEM((ns,), jnp.float32),
            pltpu.VMEM((ns,), jnp.float32),
            pltpu.VMEM((_K * qpw,), jnp.int32),
            pltpu.VMEM((qpw,), jnp.float32),
            pltpu.VMEM((qpw,), jnp.float32),
            pltpu.VMEM((qpw,), jnp.float32),
            pltpu.VMEM((qpw,), jnp.float32),
            pltpu.VMEM((qpw,), jnp.float32),
        ],
        compiler_params=pltpu.CompilerParams(needs_layout_passes=False),
    )
    signed = sc(points_a[:, 3], points_a[:, 4], points_a[:, 5],
                c_row.reshape(ns), idx_f,
                points_b[:, 0], points_b[:, 1], points_b[:, 2],
                dist[:, 0])
    return signed, idx


def kernel(points_a, points_b, k):
    del k
    return _run(points_a, points_b)


# speedup vs baseline: 1.2477x; 1.0543x over previous
import functools

import jax
import jax.numpy as jnp
from jax import lax
from jax.experimental import pallas as pl
from jax.experimental.pallas import tpu as pltpu
from jax.experimental.pallas import tpu_sc as plsc

_BQ = 128
_K = 10
_BIG_I = 2**30
_BIG_F = 1e30


def _knn_kernel(q_ref, s_ref, dist_ref, idx_ref, c_ref, *, ns):
    q = q_ref[...]
    s = s_ref[...]
    sx = s[0:3, :]
    sn = s[3:6, :]
    r_s = jnp.sum(sx * sx, axis=0, keepdims=True)
    c_ref[...] = jnp.sum(sn * sx, axis=0, keepdims=True)

    lane_mask = (lax.broadcasted_iota(jnp.int32, (1, 8), 1) < 3)
    q_xyz8 = jnp.where(lane_mask, q, 0.0)
    r_q = jnp.sum(q_xyz8 * q_xyz8, axis=1, keepdims=True)

    mul = lax.dot_general(
        q_xyz8, s, (((1,), (0,)), ((), ())),
        preferred_element_type=jnp.float32,
        precision=lax.Precision.DEFAULT)
    d2 = r_q - 2.0 * mul + r_s

    iota = lax.broadcasted_iota(jnp.int32, (_BQ, ns), 1)

    d0 = jnp.min(d2, axis=1, keepdims=True)
    idx_cols = []
    for t in range(_K):
        ct = jnp.argmin(d2, axis=1).astype(jnp.int32)[:, None]
        idx_cols.append(ct)
        if t < _K - 1:
            d2 = jnp.where(iota == ct, _BIG_F, d2)

    dist_ref[...] = jnp.sqrt(jnp.maximum(d0, 1e-12))
    idx_ref[...] = jnp.concatenate(idx_cols, axis=1)


def _sc_sign_kernel(nx_hbm, ny_hbm, nz_hbm, c_hbm, idxf_hbm, qx_hbm,
                    qy_hbm, qz_hbm, dist_hbm, out_hbm,
                    nx_v, ny_v, nz_v, c_v, idx_v, qx_v, qy_v, qz_v,
                    dist_v, out_v, *, nc, nw, nq):
    qpw = nq // nw
    wid = lax.axis_index("s") * nc + lax.axis_index("c")
    base = wid * qpw

    pltpu.sync_copy(nx_hbm, nx_v)
    pltpu.sync_copy(ny_hbm, ny_v)
    pltpu.sync_copy(nz_hbm, nz_v)
    pltpu.sync_copy(c_hbm, c_v)
    for j in range(_K):
        pltpu.sync_copy(idxf_hbm.at[pl.ds(j * nq + base, qpw)],
                        idx_v.at[pl.ds(j * qpw, qpw)])
    pltpu.sync_copy(qx_hbm.at[pl.ds(base, qpw)], qx_v)
    pltpu.sync_copy(qy_hbm.at[pl.ds(base, qpw)], qy_v)
    pltpu.sync_copy(qz_hbm.at[pl.ds(base, qpw)], qz_v)
    pltpu.sync_copy(dist_hbm.at[pl.ds(base, qpw)], dist_v)

    for i in range(qpw // 16):
        sl = pl.ds(i * 16, 16)
        qx = qx_v[sl]
        qy = qy_v[sl]
        qz = qz_v[sl]
        count = jnp.zeros((16,), jnp.int32)
        for j in range(_K):
            iv = idx_v[pl.ds(j * qpw + i * 16, 16)]
            gnx = plsc.load_gather(nx_v, [iv])
            gny = plsc.load_gather(ny_v, [iv])
            gnz = plsc.load_gather(nz_v, [iv])
            gc = plsc.load_gather(c_v, [iv])
            val = gc - (gnx * qx + gny * qy + gnz * qz)
            count = count + jnp.where(val > 0.0, 1, 0)
        dist = dist_v[sl]
        signed = jnp.where(count > 8, -dist, dist)
        out_v[sl] = jnp.minimum(qz, signed)
    pltpu.sync_copy(out_v, out_hbm.at[pl.ds(base, qpw)])


@jax.jit
def _run(points_a, points_b):
    ns = points_a.shape[0]
    nq = points_b.shape[0]
    s = jnp.zeros((8, ns), jnp.float32).at[0:6, :].set(points_a.T)
    q = jnp.zeros((nq, 8), jnp.float32).at[:, 0:6].set(points_b)
    grid = nq // _BQ
    dist, idx, c_row = pl.pallas_call(
        functools.partial(_knn_kernel, ns=ns),
        grid=(grid,),
        in_specs=[
            pl.BlockSpec((_BQ, 8), lambda i: (i, 0)),
            pl.BlockSpec((8, ns), lambda i: (0, 0)),
        ],
        out_specs=[
            pl.BlockSpec((_BQ, 1), lambda i: (i, 0)),
            pl.BlockSpec((_BQ, _K), lambda i: (i, 0)),
            pl.BlockSpec((1, ns), lambda i: (0, 0)),
        ],
        out_shape=[
            jax.ShapeDtypeStruct((nq, 1), jnp.float32),
            jax.ShapeDtypeStruct((nq, _K), jnp.int32),
            jax.ShapeDtypeStruct((1, ns), jnp.float32),
        ],
        compiler_params=pltpu.CompilerParams(
            dimension_semantics=("parallel",)),
    )(q, s)

    info = plsc.get_sparse_core_info()
    nw = info.num_cores * info.num_subcores
    qpw = nq // nw
    idx_f = idx.T.reshape(-1)
    mesh = plsc.VectorSubcoreMesh(core_axis_name="c", subcore_axis_name="s")
    sc = pl.kernel(
        functools.partial(_sc_sign_kernel, nc=info.num_cores, nw=nw, nq=nq),
        mesh=mesh,
        out_type=jax.ShapeDtypeStruct((nq,), jnp.float32),
        scratch_types=[
            pltpu.VMEM((ns,), jnp.float32),
            pltpu.VMEM((ns,), jnp.float32),
            pltpu.VMEM((ns,), jnp.float32),
            pltpu.VMEM((ns,), jnp.float32),
            pltpu.VMEM((_K * qpw,), jnp.int32),
            pltpu.VMEM((qpw,), jnp.float32),
            pltpu.VMEM((qpw,), jnp.float32),
            pltpu.VMEM((qpw,), jnp.float32),
            pltpu.VMEM((qpw,), jnp.float32),
            pltpu.VMEM((qpw,), jnp.float32),
        ],
        compiler_params=pltpu.CompilerParams(needs_layout_passes=False),
    )
    signed = sc(points_a[:, 3], points_a[:, 4], points_a[:, 5],
                c_row.reshape(ns), idx_f,
                points_b[:, 0], points_b[:, 1], points_b[:, 2],
                dist[:, 0])
    return signed, idx


def kernel(points_a, points_b, k):
    del k
    return _run(points_a, points_b)
